# baseline (device time: 641248 ns/iter reference)
import jax
import jax.numpy as jnp
from jax import lax
from jax.experimental import pallas as pl
from jax.experimental.pallas import tpu as pltpu

N_DEV = 4
SQ = 256
SKV = 4096
D_MODEL = 1024
H_PER = 8
P_PER = 4
DH = 128
PW = 2 * DH
NB = 64
SCALE = 0.08838834764831843

NA = 22
NBB = 23
NC = 23


def _body(x_ref, wq_ref, kf_hbm, vf_hbm, wo_ref, out_ref,
          xg, kA, kB, kC, vA, vB, vC, comm, sbuf,
          ag_send, ag_recv, rs_send, rs_recv, ksems, vsems):
    my = lax.axis_index("i")
    left = (my - 1) % N_DEV
    right = (my + 1) % N_DEV

    barrier_sem = pltpu.get_barrier_semaphore()
    for nbr in [left, right]:
        pl.semaphore_signal(
            barrier_sem, inc=1,
            device_id=(nbr,), device_id_type=pl.DeviceIdType.MESH,
        )
    pl.semaphore_wait(barrier_sem, 2)

    xg[my] = x_ref[0].astype(jnp.bfloat16)

    def ag_hop(h):
        o = (my - h) % N_DEV
        rdma = pltpu.make_async_remote_copy(
            src_ref=xg.at[o], dst_ref=xg.at[o],
            send_sem=ag_send.at[h], recv_sem=ag_recv.at[h],
            device_id=(right,), device_id_type=pl.DeviceIdType.MESH,
        )
        rdma.start()
        return rdma

    def rs_hop(s):
        rdma = pltpu.make_async_remote_copy(
            src_ref=sbuf.at[s], dst_ref=comm.at[s],
            send_sem=rs_send.at[s], recv_sem=rs_recv.at[s],
            device_id=(right,), device_id_type=pl.DeviceIdType.MESH,
        )
        rdma.start()
        return rdma

    batches = [my, (my - 1) % N_DEV, (my - 2) % N_DEV, (my + 1) % N_DEV]

    def _upos(u):
        bi, p = divmod(u, P_PER)
        slot = u % 2
        hgp = my * P_PER + p
        return batches[bi], hgp, slot

    def _family_copies(i, u):
        b, hgp, slot = _upos(u)
        cs = []
        for sf, bufs, sems in ((kf_hbm, (kA, kB, kC), ksems),
                               (vf_hbm, (vA, vB, vC), vsems)):
            bufA, bufB, bufC = bufs
            cs.append(pltpu.make_async_copy(
                sf.at[b, 3 * i, :, hgp, :], bufA.at[slot, i],
                sems.at[slot, 0]))
            cs.append(pltpu.make_async_copy(
                sf.at[b, 3 * i + 2, :, hgp, :], bufB.at[slot, 2 + i],
                sems.at[slot, 1]))
            cs.append(pltpu.make_async_copy(
                sf.at[b, 3 * i + 1, :, hgp, :], bufC.at[slot, 2 + i],
                sems.at[slot, 2]))
        return cs

    def _extra_copies(u, do_start):
        b, hgp, slot = _upos(u)
        cs = []
        for sf, bufs, sems in ((kf_hbm, (kA, kB, kC), ksems),
                               (vf_hbm, (vA, vB, vC), vsems)):
            bufA, bufB, bufC = bufs
            cs.append(pltpu.make_async_copy(
                sf.at[b, 63, :, hgp, :], bufA.at[slot, 21], sems.at[slot, 0]))
            cs.append(pltpu.make_async_copy(
                sf.at[b, pl.ds(0, 2), :, hgp, :], bufB.at[slot, pl.ds(0, 2)],
                sems.at[slot, 1]))
            cs.append(pltpu.make_async_copy(
                sf.at[b, 0, :, hgp, :], bufC.at[slot, 0], sems.at[slot, 2]))
            cs.append(pltpu.make_async_copy(
                sf.at[b, 2, :, hgp, :], bufC.at[slot, 1], sems.at[slot, 2]))
        for c in cs:
            if do_start:
                c.start()
            else:
                c.wait()

    def start_copies(u):
        def go(i, carry):
            for c in _family_copies(i, u):
                c.start()
            return carry
        lax.fori_loop(0, 21, go, 0)
        _extra_copies(u, True)

    def wait_copies(u):
        def go(i, carry):
            for c in _family_copies(i, u):
                c.wait()
            return carry
        lax.fori_loop(0, 21, go, 0)
        _extra_copies(u, False)

    n_pairs = N_DEV * P_PER
    start_copies(0)

    def pair_step(u, xp16):
        _, p = divmod(u, P_PER)
        slot = u % 2
        if u + 1 < n_pairs:
            start_copies(u + 1)
        wait_copies(u)
        total = None
        for hh in range(2):
            h = 2 * p + hh
            q = jnp.dot(xp16, wq_ref[:, h * DH:(h + 1) * DH],
                        preferred_element_type=jnp.float32)
            q16 = q.astype(jnp.bfloat16)

            def piece(qrows, kbuf, vbuf, nblk):
                k16 = kbuf[slot][:, :, hh * DH:(hh + 1) * DH].reshape(
                    nblk * 64, DH).astype(jnp.bfloat16)
                s = lax.dot_general(qrows, k16, (((1,), (1,)), ((), ())),
                                    preferred_element_type=jnp.float32)
                e = jnp.exp(s * SCALE)
                d = jnp.sum(e, axis=1, keepdims=True)
                v16 = vbuf[slot][:, :, hh * DH:(hh + 1) * DH].reshape(
                    nblk * 64, DH).astype(jnp.bfloat16)
                ctx = jnp.dot(e.astype(jnp.bfloat16), v16,
                              preferred_element_type=jnp.float32)
                return ctx / d

            ctxA = piece(q16[0:128], kA, vA, NA)
            ctxB = piece(q16[128:192], kB, vB, NBB)
            ctxC = piece(q16[192:256], kC, vC, NC)
            ctx = jnp.concatenate(
                [ctxA[0:64], ctxB, ctxC, ctxA[64:128]], axis=0)
            po = jnp.dot(ctx.astype(jnp.bfloat16),
                         wo_ref[h * DH:(h + 1) * DH, :],
                         preferred_element_type=jnp.float32)
            total = po if total is None else total + po
        return total

    def batch_partial(bi):
        x16 = xg[batches[bi]]
        xp16 = jnp.concatenate(
            [x16[0:64], x16[192:256], x16[64:128], x16[128:192]], axis=0)
        acc = None
        for p in range(P_PER):
            po = pair_step(bi * P_PER + p, xp16)
            acc = po if acc is None else acc + po
        return acc

    ag = [None] * (N_DEV - 1)
    rs = [None] * (N_DEV - 1)

    ag[0] = ag_hop(0)
    p_own = batch_partial(0)

    ag[0].wait_recv()
    ag[1] = ag_hop(1)
    p1 = batch_partial(1)
    sbuf[0] = p1.astype(jnp.bfloat16)
    rs[0] = rs_hop(0)

    ag[1].wait_recv()
    ag[2] = ag_hop(2)
    p2 = batch_partial(2)
    rs[0].wait_recv()
    sbuf[1] = (comm[0].astype(jnp.float32) + p2).astype(jnp.bfloat16)
    rs[1] = rs_hop(1)

    ag[2].wait_recv()
    p3 = batch_partial(3)
    rs[1].wait_recv()
    sbuf[2] = (comm[1].astype(jnp.float32) + p3).astype(jnp.bfloat16)
    rs[2] = rs_hop(2)

    rs[2].wait_recv()
    out_ref[0] = comm[2].astype(jnp.float32) + p_own

    for r in ag + rs:
        r.wait_send()


def kernel(x, Wq, K_ext, V_ext, Wo):
    kf = K_ext.reshape(N_DEV, NB, 64, 16, PW)
    vf = V_ext.reshape(N_DEV, NB, 64, 16, PW)
    return pl.pallas_call(
        _body,
        out_shape=jax.ShapeDtypeStruct((1, SQ, D_MODEL), jnp.float32),
        in_specs=[
            pl.BlockSpec(memory_space=pltpu.VMEM),
            pl.BlockSpec(memory_space=pltpu.VMEM),
            pl.BlockSpec(memory_space=pl.ANY),
            pl.BlockSpec(memory_space=pl.ANY),
            pl.BlockSpec(memory_space=pltpu.VMEM),
        ],
        out_specs=pl.BlockSpec(memory_space=pltpu.VMEM),
        scratch_shapes=[
            pltpu.VMEM((N_DEV, SQ, D_MODEL), jnp.bfloat16),
            pltpu.VMEM((2, NA, 64, PW), jnp.float32),
            pltpu.VMEM((2, NBB, 64, PW), jnp.float32),
            pltpu.VMEM((2, NC, 64, PW), jnp.float32),
            pltpu.VMEM((2, NA, 64, PW), jnp.float32),
            pltpu.VMEM((2, NBB, 64, PW), jnp.float32),
            pltpu.VMEM((2, NC, 64, PW), jnp.float32),
            pltpu.VMEM((N_DEV - 1, SQ, D_MODEL), jnp.bfloat16),
            pltpu.VMEM((N_DEV - 1, SQ, D_MODEL), jnp.bfloat16),
            pltpu.SemaphoreType.DMA((N_DEV - 1,)),
            pltpu.SemaphoreType.DMA((N_DEV - 1,)),
            pltpu.SemaphoreType.DMA((N_DEV - 1,)),
            pltpu.SemaphoreType.DMA((N_DEV - 1,)),
            pltpu.SemaphoreType.DMA((2, 3)),
            pltpu.SemaphoreType.DMA((2, 3)),
        ],
        compiler_params=pltpu.CompilerParams(collective_id=0),
    )(x, Wq.astype(jnp.bfloat16), kf, vf, Wo.astype(jnp.bfloat16))


# device time: 88811 ns/iter; 7.2204x vs baseline; 7.2204x over previous
import jax
import jax.numpy as jnp
from jax import lax
from jax.experimental import pallas as pl
from jax.experimental.pallas import tpu as pltpu

N_DEV = 4
SQ = 256
SKV = 4096
D_MODEL = 1024
H_PER = 8
DH = 128
SCALE = 0.08838834764831843


def _body(x_ref, wq_ref, k_hbm, v_hbm, wo_ref, out_ref,
          xg, kbuf, vbuf, comm, sbuf,
          ag_send, ag_recv, rs_send, rs_recv, ksems, vsems):
    my = lax.axis_index("i")
    left = (my - 1) % N_DEV
    right = (my + 1) % N_DEV

    barrier_sem = pltpu.get_barrier_semaphore()
    for nbr in [left, right]:
        pl.semaphore_signal(
            barrier_sem, inc=1,
            device_id=(nbr,), device_id_type=pl.DeviceIdType.MESH,
        )
    pl.semaphore_wait(barrier_sem, 2)

    row = lax.broadcasted_iota(jnp.int32, (SQ, SKV), 0)
    col = lax.broadcasted_iota(jnp.int32, (SQ, SKV), 1)
    qb = row // 64
    kb = col // 64
    mask = (qb == kb) | (kb == 0) | (((qb + kb) % 3) == 0)
    bias = jnp.where(mask, 0.0, -1e9).astype(jnp.float32)

    xg[my] = x_ref[0].astype(jnp.bfloat16)

    def ag_hop(h):
        o = (my - h) % N_DEV
        rdma = pltpu.make_async_remote_copy(
            src_ref=xg.at[o], dst_ref=xg.at[o],
            send_sem=ag_send.at[h], recv_sem=ag_recv.at[h],
            device_id=(right,), device_id_type=pl.DeviceIdType.MESH,
        )
        rdma.start()
        return rdma

    def rs_hop(s):
        rdma = pltpu.make_async_remote_copy(
            src_ref=sbuf.at[s], dst_ref=comm.at[s],
            send_sem=rs_send.at[s], recv_sem=rs_recv.at[s],
            device_id=(right,), device_id_type=pl.DeviceIdType.MESH,
        )
        rdma.start()
        return rdma

    batches = [my, (my - 1) % N_DEV, (my - 2) % N_DEV, (my + 1) % N_DEV]

    def start_copies(t):
        bi, h = divmod(t, H_PER)
        slot = t % 2
        hg = my * H_PER + h
        b = batches[bi]
        kc = pltpu.make_async_copy(k_hbm.at[b, :, hg, :], kbuf.at[slot],
                                   ksems.at[slot])
        vc = pltpu.make_async_copy(v_hbm.at[b, :, hg, :], vbuf.at[slot],
                                   vsems.at[slot])
        kc.start()
        vc.start()
        return kc, vc

    n_steps = N_DEV * H_PER
    copies = [None] * n_steps
    copies[0] = start_copies(0)

    def head_step(t, x16):
        _, h = divmod(t, H_PER)
        slot = t % 2
        if t + 1 < n_steps:
            copies[t + 1] = start_copies(t + 1)
        kc, vc = copies[t]
        kc.wait()
        vc.wait()
        q = jnp.dot(x16, wq_ref[:, h * DH:(h + 1) * DH],
                    preferred_element_type=jnp.float32)
        q16 = q.astype(jnp.bfloat16)
        k16 = kbuf[slot].astype(jnp.bfloat16)
        s = lax.dot_general(q16, k16, (((1,), (1,)), ((), ())),
                            preferred_element_type=jnp.float32)
        s = s + bias
        e = jnp.exp(s)
        denom = jnp.sum(e, axis=1, keepdims=True)
        e16 = e.astype(jnp.bfloat16)
        v16 = vbuf[slot].astype(jnp.bfloat16)
        ctx = jnp.dot(e16, v16, preferred_element_type=jnp.float32)
        ctx = ctx / denom
        return jnp.dot(ctx.astype(jnp.bfloat16),
                       wo_ref[h * DH:(h + 1) * DH, :],
                       preferred_element_type=jnp.float32)

    def batch_partial(bi):
        x16 = xg[batches[bi]]
        acc = None
        for h in range(H_PER):
            po = head_step(bi * H_PER + h, x16)
            acc = po if acc is None else acc + po
        return acc

    ag = [None] * (N_DEV - 1)
    rs = [None] * (N_DEV - 1)

    ag[0] = ag_hop(0)
    p_own = batch_partial(0)

    ag[0].wait_recv()
    ag[1] = ag_hop(1)
    p1 = batch_partial(1)
    sbuf[0] = p1.astype(jnp.bfloat16)
    rs[0] = rs_hop(0)

    ag[1].wait_recv()
    ag[2] = ag_hop(2)
    p2 = batch_partial(2)
    rs[0].wait_recv()
    sbuf[1] = (comm[0].astype(jnp.float32) + p2).astype(jnp.bfloat16)
    rs[1] = rs_hop(1)

    ag[2].wait_recv()
    p3 = batch_partial(3)
    rs[1].wait_recv()
    sbuf[2] = (comm[1].astype(jnp.float32) + p3).astype(jnp.bfloat16)
    rs[2] = rs_hop(2)

    rs[2].wait_recv()
    out_ref[0] = comm[2].astype(jnp.float32) + p_own

    for r in ag + rs:
        r.wait_send()


def kernel(x, Wq, K_ext, V_ext, Wo):
    return pl.pallas_call(
        _body,
        out_shape=jax.ShapeDtypeStruct((1, SQ, D_MODEL), jnp.float32),
        in_specs=[
            pl.BlockSpec(memory_space=pltpu.VMEM),
            pl.BlockSpec(memory_space=pltpu.VMEM),
            pl.BlockSpec(memory_space=pl.ANY),
            pl.BlockSpec(memory_space=pl.ANY),
            pl.BlockSpec(memory_space=pltpu.VMEM),
        ],
        out_specs=pl.BlockSpec(memory_space=pltpu.VMEM),
        scratch_shapes=[
            pltpu.VMEM((N_DEV, SQ, D_MODEL), jnp.bfloat16),
            pltpu.VMEM((2, SKV, DH), jnp.float32),
            pltpu.VMEM((2, SKV, DH), jnp.float32),
            pltpu.VMEM((N_DEV - 1, SQ, D_MODEL), jnp.bfloat16),
            pltpu.VMEM((N_DEV - 1, SQ, D_MODEL), jnp.bfloat16),
            pltpu.SemaphoreType.DMA((N_DEV - 1,)),
            pltpu.SemaphoreType.DMA((N_DEV - 1,)),
            pltpu.SemaphoreType.DMA((N_DEV - 1,)),
            pltpu.SemaphoreType.DMA((N_DEV - 1,)),
            pltpu.SemaphoreType.DMA((2,)),
            pltpu.SemaphoreType.DMA((2,)),
        ],
        compiler_params=pltpu.CompilerParams(collective_id=0),
    )(x, (Wq * SCALE).astype(jnp.bfloat16), K_ext, V_ext,
      Wo.astype(jnp.bfloat16))


# device time: 87412 ns/iter; 7.3359x vs baseline; 1.0160x over previous
import jax
import jax.numpy as jnp
from jax import lax
from jax.experimental import pallas as pl
from jax.experimental.pallas import tpu as pltpu

N_DEV = 4
SQ = 256
SKV = 4096
D_MODEL = 1024
H_PER = 8
DH = 128
SCALE = 0.08838834764831843


def _body(x_ref, wq_ref, k_hbm, v_hbm, wo_ref, out_ref,
          xg, kbuf, vbuf, comm, sbuf,
          ag_send, ag_recv, rs_send, rs_recv, ksems, vsems):
    my = lax.axis_index("i")
    left = (my - 1) % N_DEV
    right = (my + 1) % N_DEV

    barrier_sem = pltpu.get_barrier_semaphore()
    for nbr in [left, right]:
        pl.semaphore_signal(
            barrier_sem, inc=1,
            device_id=(nbr,), device_id_type=pl.DeviceIdType.MESH,
        )
    pl.semaphore_wait(barrier_sem, 2)

    row = lax.broadcasted_iota(jnp.int32, (SQ, SKV), 0)
    col = lax.broadcasted_iota(jnp.int32, (SQ, SKV), 1)
    qb = row // 64
    kb = col // 64
    mask = (qb == kb) | (kb == 0) | (((qb + kb) % 3) == 0)
    bias = jnp.where(mask, 0.0, -1e9).astype(jnp.float32)

    xg[my] = x_ref[0].astype(jnp.bfloat16)

    def ag_hop(h):
        o = (my - h) % N_DEV
        rdma = pltpu.make_async_remote_copy(
            src_ref=xg.at[o], dst_ref=xg.at[o],
            send_sem=ag_send.at[h], recv_sem=ag_recv.at[h],
            device_id=(right,), device_id_type=pl.DeviceIdType.MESH,
        )
        rdma.start()
        return rdma

    def rs_hop(s):
        rdma = pltpu.make_async_remote_copy(
            src_ref=sbuf.at[s], dst_ref=comm.at[s],
            send_sem=rs_send.at[s], recv_sem=rs_recv.at[s],
            device_id=(right,), device_id_type=pl.DeviceIdType.MESH,
        )
        rdma.start()
        return rdma

    batches = [my, (my - 1) % N_DEV, (my - 2) % N_DEV, (my + 1) % N_DEV]

    def start_copies(t):
        bi, h = divmod(t, H_PER)
        slot = t % 2
        hg = my * H_PER + h
        b = batches[bi]
        kc = pltpu.make_async_copy(k_hbm.at[b, :, hg, :], kbuf.at[slot],
                                   ksems.at[slot])
        vc = pltpu.make_async_copy(v_hbm.at[b, :, hg, :], vbuf.at[slot],
                                   vsems.at[slot])
        kc.start()
        vc.start()
        return kc, vc

    n_steps = N_DEV * H_PER
    copies = [None] * n_steps
    copies[0] = start_copies(0)

    def head_step(t, x16):
        _, h = divmod(t, H_PER)
        slot = t % 2
        if t + 1 < n_steps:
            copies[t + 1] = start_copies(t + 1)
        kc, vc = copies[t]
        kc.wait()
        vc.wait()
        q = jnp.dot(x16, wq_ref[:, h * DH:(h + 1) * DH],
                    preferred_element_type=jnp.float32)
        q16 = q.astype(jnp.bfloat16)
        k16 = kbuf[slot].astype(jnp.bfloat16)
        s = lax.dot_general(q16, k16, (((1,), (1,)), ((), ())),
                            preferred_element_type=jnp.float32)
        s = s + bias
        e = jnp.exp2(s)
        denom = jnp.sum(e, axis=1, keepdims=True)
        e16 = e.astype(jnp.bfloat16)
        v16 = vbuf[slot].astype(jnp.bfloat16)
        ctx = jnp.dot(e16, v16, preferred_element_type=jnp.float32)
        ctx = ctx / denom
        return jnp.dot(ctx.astype(jnp.bfloat16),
                       wo_ref[h * DH:(h + 1) * DH, :],
                       preferred_element_type=jnp.float32)

    def batch_partial(bi):
        x16 = xg[batches[bi]]
        acc = None
        for h in range(H_PER):
            po = head_step(bi * H_PER + h, x16)
            acc = po if acc is None else acc + po
        return acc

    ag = [None] * (N_DEV - 1)
    rs = [None] * (N_DEV - 1)

    ag[0] = ag_hop(0)
    p_own = batch_partial(0)

    ag[0].wait_recv()
    ag[1] = ag_hop(1)
    p1 = batch_partial(1)
    sbuf[0] = p1.astype(jnp.bfloat16)
    rs[0] = rs_hop(0)

    ag[1].wait_recv()
    ag[2] = ag_hop(2)
    p2 = batch_partial(2)
    rs[0].wait_recv()
    sbuf[1] = (comm[0].astype(jnp.float32) + p2).astype(jnp.bfloat16)
    rs[1] = rs_hop(1)

    ag[2].wait_recv()
    p3 = batch_partial(3)
    rs[1].wait_recv()
    sbuf[2] = (comm[1].astype(jnp.float32) + p3).astype(jnp.bfloat16)
    rs[2] = rs_hop(2)

    rs[2].wait_recv()
    out_ref[0] = comm[2].astype(jnp.float32) + p_own

    for r in ag + rs:
        r.wait_send()


def kernel(x, Wq, K_ext, V_ext, Wo):
    return pl.pallas_call(
        _body,
        out_shape=jax.ShapeDtypeStruct((1, SQ, D_MODEL), jnp.float32),
        in_specs=[
            pl.BlockSpec(memory_space=pltpu.VMEM),
            pl.BlockSpec(memory_space=pltpu.VMEM),
            pl.BlockSpec(memory_space=pl.ANY),
            pl.BlockSpec(memory_space=pl.ANY),
            pl.BlockSpec(memory_space=pltpu.VMEM),
        ],
        out_specs=pl.BlockSpec(memory_space=pltpu.VMEM),
        scratch_shapes=[
            pltpu.VMEM((N_DEV, SQ, D_MODEL), jnp.bfloat16),
            pltpu.VMEM((2, SKV, DH), jnp.float32),
            pltpu.VMEM((2, SKV, DH), jnp.float32),
            pltpu.VMEM((N_DEV - 1, SQ, D_MODEL), jnp.bfloat16),
            pltpu.VMEM((N_DEV - 1, SQ, D_MODEL), jnp.bfloat16),
            pltpu.SemaphoreType.DMA((N_DEV - 1,)),
            pltpu.SemaphoreType.DMA((N_DEV - 1,)),
            pltpu.SemaphoreType.DMA((N_DEV - 1,)),
            pltpu.SemaphoreType.DMA((N_DEV - 1,)),
            pltpu.SemaphoreType.DMA((2,)),
            pltpu.SemaphoreType.DMA((2,)),
        ],
        compiler_params=pltpu.CompilerParams(collective_id=0),
    )(x, (Wq * (SCALE * 1.4426950408889634)).astype(jnp.bfloat16),
      K_ext, V_ext, Wo.astype(jnp.bfloat16))
